# Initial kernel scaffold; baseline (speedup 1.0000x reference)
import jax
import jax.numpy as jnp
from jax.experimental import pallas as pl


def kernel(log2_0, log2_1, log2_2, log2_3, log2_4, log2_5, log10_0, log10_1, log10_2, log10_3, log10_4, log10_5, log10_6, sparse_0, sparse_1, sparse_2, sparse_3, sparse_4, sparse_5, sparse_6, sparse_7, sparse_8, sparse_9, sparse_10, sparse_11, sparse_12, sparse_13, sparse_14, sparse_15, sparse_16, sparse_17, sparse_18, sparse_19, sparse_20, sparse_21, sparse_22, sparse_23, beh_0, beh_1, emb_sparse_0, emb_sparse_1, emb_sparse_2, emb_sparse_3, emb_sparse_4, emb_sparse_5, emb_sparse_6, emb_sparse_7, emb_sparse_8, emb_sparse_9, emb_sparse_10, emb_sparse_11, emb_sparse_12, emb_sparse_13, emb_sparse_14, emb_sparse_15, emb_sparse_16, emb_sparse_17, emb_sparse_18, emb_sparse_19, emb_sparse_20, emb_sparse_21, emb_sparse_22, emb_sparse_23, emb_beh_0, emb_beh_1):
    raise NotImplementedError("write your pallas kernel here")


# speedup vs baseline: 75.9890x; 75.9890x over previous
import functools
import math

import jax
import jax.numpy as jnp
from jax import lax
from jax.experimental import pallas as pl
from jax.experimental.pallas import tpu as pltpu
from jax.experimental.pallas import tpu_sc as plsc

B = 16384
NUM_BINS = 1000
EMB_DIM = 8
HIST = 200
N_SPARSE = 24
N_COLS = 13 + 26 * EMB_DIM
NUM_WORKERS = 32
ROWS_PER_WORKER = B // NUM_WORKERS
R = 64
N_CHUNKS = ROWS_PER_WORKER // R
L = 16


def _numeric_tc(xpad, scale_row):
    def body(x_ref, s_ref, o_ref):
        o_ref[...] = jnp.log1p(x_ref[...]) * s_ref[...]
    return pl.pallas_call(
        body,
        out_shape=jax.ShapeDtypeStruct(xpad.shape, jnp.float32),
    )(xpad, scale_row)


def _sc_body(num_hbm, spi_hbm, b0_hbm, b1_hbm, t0_hbm, t1_hbm, *rest):
    embs = rest[:N_SPARSE]
    out_hbm = rest[N_SPARSE]
    (tbl0_v, tbl1_v, num_v, beh0_v, beh1_v, spi_v, gbuf, out_v, sem) = \
        rest[N_SPARSE + 1:]

    wid = lax.axis_index("s") * 2 + lax.axis_index("c")
    base_row = wid * ROWS_PER_WORKER

    iota = lax.iota(jnp.int32, L)
    iota200 = iota * 200
    iota221 = iota * 221
    m13 = iota < 13
    lane_r = lax.shift_right_logical(iota, 3)
    lane_d = jnp.bitwise_and(iota, 7)
    pat221 = lane_r * 221 + lane_d
    cd = [jnp.full((L,), d, jnp.int32) for d in range(EMB_DIM)]
    zero = jnp.zeros((L,), jnp.float32)

    pltpu.sync_copy(t0_hbm, tbl0_v)
    pltpu.sync_copy(t1_hbm, tbl1_v)
    for k in range(N_SPARSE):
        pltpu.sync_copy(
            spi_hbm.at[pl.ds(k * B + base_row, ROWS_PER_WORKER)],
            spi_v.at[pl.ds(k * ROWS_PER_WORKER, ROWS_PER_WORKER)])

    def chunk_body(chunk, carry):
        row0 = base_row + chunk * R

        copies = []
        for k in range(N_SPARSE):
            idx_ref = spi_v.at[pl.ds(k * ROWS_PER_WORKER + chunk * R, R)]
            copies.append(pltpu.async_copy(embs[k].at[idx_ref], gbuf.at[k], sem))

        pltpu.sync_copy(num_hbm.at[pl.ds(row0 * L, R * L)], num_v)
        pltpu.sync_copy(b0_hbm.at[pl.ds(row0 * HIST, R * HIST)], beh0_v)
        pltpu.sync_copy(b1_hbm.at[pl.ds(row0 * HIST, R * HIST)], beh1_v)

        def nbody(r, c):
            src = plsc.load_gather(num_v, [iota + r * L])
            plsc.store_scatter(out_v, [iota + r * 221], src, mask=m13)
            return c
        lax.fori_loop(0, R, nbody, 0)

        for g in range(R // L):
            pos_base = iota200 + g * L * HIST
            dst_base = iota221 + g * L * 221
            for coff, beh_v, tbl_v in ((205, beh0_v, tbl0_v),
                                       (213, beh1_v, tbl1_v)):
                def hbody(h, accs, beh_v=beh_v, tbl_v=tbl_v, pos_base=pos_base):
                    idx = plsc.load_gather(beh_v, [pos_base + h])
                    return tuple(
                        accs[d] + plsc.load_gather(tbl_v, [idx, cd[d]])
                        for d in range(EMB_DIM))
                accs = lax.fori_loop(0, HIST, hbody, (zero,) * EMB_DIM)
                for d in range(EMB_DIM):
                    plsc.store_scatter(out_v, [dst_base + (coff + d)], accs[d])

        for c in copies:
            c.wait()
        for k in range(N_SPARSE):
            kk = jnp.full((L,), k, jnp.int32)
            def sbody(j, c, kk=kk, k=k):
                src = plsc.load_gather(gbuf, [kk, lane_r + j * 2, lane_d])
                plsc.store_scatter(out_v, [pat221 + (j * 442 + 13 + 8 * k)], src)
                return c
            lax.fori_loop(0, R // 2, sbody, 0)

        pltpu.sync_copy(out_v, out_hbm.at[pl.ds(row0 * N_COLS, R * N_COLS)])
        return carry

    lax.fori_loop(0, N_CHUNKS, chunk_body, 0)


_sc_kernel = functools.partial(
    pl.kernel,
    out_type=jax.ShapeDtypeStruct((B * N_COLS,), jnp.float32),
    mesh=plsc.VectorSubcoreMesh(core_axis_name="c", subcore_axis_name="s"),
    compiler_params=pltpu.CompilerParams(needs_layout_passes=False,
                                         use_tc_tiling_on_sc=False),
    scratch_types=[
        pltpu.VMEM((NUM_BINS, EMB_DIM), jnp.float32),
        pltpu.VMEM((NUM_BINS, EMB_DIM), jnp.float32),
        pltpu.VMEM((R * L,), jnp.float32),
        pltpu.VMEM((R * HIST,), jnp.int32),
        pltpu.VMEM((R * HIST,), jnp.int32),
        pltpu.VMEM((N_SPARSE * ROWS_PER_WORKER,), jnp.int32),
        pltpu.VMEM((N_SPARSE, R, EMB_DIM), jnp.float32),
        pltpu.VMEM((R * N_COLS,), jnp.float32),
        pltpu.SemaphoreType.DMA,
    ],
)(_sc_body)


def kernel(log2_0, log2_1, log2_2, log2_3, log2_4, log2_5, log10_0, log10_1, log10_2, log10_3, log10_4, log10_5, log10_6, sparse_0, sparse_1, sparse_2, sparse_3, sparse_4, sparse_5, sparse_6, sparse_7, sparse_8, sparse_9, sparse_10, sparse_11, sparse_12, sparse_13, sparse_14, sparse_15, sparse_16, sparse_17, sparse_18, sparse_19, sparse_20, sparse_21, sparse_22, sparse_23, beh_0, beh_1, emb_sparse_0, emb_sparse_1, emb_sparse_2, emb_sparse_3, emb_sparse_4, emb_sparse_5, emb_sparse_6, emb_sparse_7, emb_sparse_8, emb_sparse_9, emb_sparse_10, emb_sparse_11, emb_sparse_12, emb_sparse_13, emb_sparse_14, emb_sparse_15, emb_sparse_16, emb_sparse_17, emb_sparse_18, emb_sparse_19, emb_sparse_20, emb_sparse_21, emb_sparse_22, emb_sparse_23, emb_beh_0, emb_beh_1):
    logs = [log2_0, log2_1, log2_2, log2_3, log2_4, log2_5,
            log10_0, log10_1, log10_2, log10_3, log10_4, log10_5, log10_6]
    sparse = [sparse_0, sparse_1, sparse_2, sparse_3, sparse_4, sparse_5,
              sparse_6, sparse_7, sparse_8, sparse_9, sparse_10, sparse_11,
              sparse_12, sparse_13, sparse_14, sparse_15, sparse_16, sparse_17,
              sparse_18, sparse_19, sparse_20, sparse_21, sparse_22, sparse_23]
    embs = [emb_sparse_0, emb_sparse_1, emb_sparse_2, emb_sparse_3,
            emb_sparse_4, emb_sparse_5, emb_sparse_6, emb_sparse_7,
            emb_sparse_8, emb_sparse_9, emb_sparse_10, emb_sparse_11,
            emb_sparse_12, emb_sparse_13, emb_sparse_14, emb_sparse_15,
            emb_sparse_16, emb_sparse_17, emb_sparse_18, emb_sparse_19,
            emb_sparse_20, emb_sparse_21, emb_sparse_22, emb_sparse_23]

    xpad = jnp.concatenate(logs + [jnp.zeros((B, 3), jnp.float32)], axis=1)
    scale16 = [1.0 / math.log(2.0)] * 6 + [1.0 / math.log(10.0)] * 7 + [0.0] * 3
    scale_row = jnp.tile(jnp.asarray(scale16, jnp.float32), 8)[None, :]
    num = _numeric_tc(xpad.reshape(B * L // 128, 128),
                      jnp.broadcast_to(scale_row, (B * L // 128, 128)))

    spi = jnp.stack([s.astype(jnp.int32).reshape(B) for s in sparse], axis=0)
    out_flat = _sc_kernel(
        num.reshape(-1),
        spi.reshape(-1),
        beh_0.astype(jnp.int32).reshape(-1),
        beh_1.astype(jnp.int32).reshape(-1),
        emb_beh_0, emb_beh_1,
        *embs,
    )
    return out_flat.reshape(B, N_COLS)


# speedup vs baseline: 83.1700x; 1.0945x over previous
import functools
import math

import jax
import jax.numpy as jnp
from jax import lax
from jax.experimental import pallas as pl
from jax.experimental.pallas import tpu as pltpu
from jax.experimental.pallas import tpu_sc as plsc

B = 16384
NUM_BINS = 1000
EMB_DIM = 8
NW = EMB_DIM // 2
HIST = 200
N_SPARSE = 24
N_COLS = 13 + 26 * EMB_DIM
NUM_WORKERS = 32
ROWS_PER_WORKER = B // NUM_WORKERS
R = 32
N_CHUNKS = ROWS_PER_WORKER // R
GIDX = 128
N_GD = ROWS_PER_WORKER // GIDX
L = 16


def _numeric_tc(xpad, scale_row):
    def body(x_ref, s_ref, o_ref):
        o_ref[...] = jnp.log1p(x_ref[...]) * s_ref[...]
    return pl.pallas_call(
        body,
        out_shape=jax.ShapeDtypeStruct(xpad.shape, jnp.float32),
    )(xpad, scale_row)


def _pack_bf16(tbl):
    h = lax.bitcast_convert_type(tbl.astype(jnp.bfloat16), jnp.uint16)
    w = h[:, 0::2].astype(jnp.uint32) | (h[:, 1::2].astype(jnp.uint32) << 16)
    return lax.bitcast_convert_type(w, jnp.int32)


def _sc_body(num_hbm, spi_hbm, b0_hbm, b1_hbm, t0_hbm, t1_hbm, *rest):
    embs = rest[:N_SPARSE]
    out_hbm = rest[N_SPARSE]
    (tbl0_v, tbl1_v, numA, numB, b0A, b0B, b1A, b1B, spi_v, gA, gB, out_v,
     semA, semB) = rest[N_SPARSE + 1:]

    wid = lax.axis_index("s") * 2 + lax.axis_index("c")
    base_row = wid * ROWS_PER_WORKER

    iota = lax.iota(jnp.int32, L)
    iotaH = iota * HIST
    iota221 = iota * 221
    m13 = iota < 13
    lane_r = lax.shift_right_logical(iota, 3)
    lane_d = jnp.bitwise_and(iota, 7)
    pat221 = lane_r * 221 + lane_d
    cw = [jnp.full((L,), w, jnp.int32) for w in range(NW)]
    zero = jnp.zeros((L,), jnp.float32)
    himask = jnp.full((L,), -65536, jnp.int32)

    def unpack(word):
        lo = plsc.bitcast(jnp.left_shift(word, 16), jnp.float32)
        hi = plsc.bitcast(jnp.bitwise_and(word, himask), jnp.float32)
        return lo, hi

    for k in range(N_SPARSE):
        pltpu.sync_copy(
            spi_hbm.at[pl.ds(k * B + base_row, ROWS_PER_WORKER)],
            spi_v.at[pl.ds(k * ROWS_PER_WORKER, ROWS_PER_WORKER)])
    def stage(chunk, bufs, sem):
        row0 = base_row + chunk * R
        pltpu.async_copy(b0_hbm.at[pl.ds(row0 * HIST, R * HIST)], bufs[0], sem)
        pltpu.async_copy(b1_hbm.at[pl.ds(row0 * HIST, R * HIST)], bufs[1], sem)
        pltpu.async_copy(num_hbm.at[pl.ds(row0 * L, R * L)], bufs[2], sem)
        for k in range(N_SPARSE):
            idx_ref = spi_v.at[pl.ds(k * ROWS_PER_WORKER + chunk * R, R)]
            pltpu.async_copy(embs[k].at[idx_ref],
                             bufs[3].at[pl.ds(k * R, R)], sem)

    def drain(bufs, sem):
        pltpu.make_async_copy(b0_hbm.at[pl.ds(0, R * HIST)], bufs[0], sem).wait()
        pltpu.make_async_copy(b1_hbm.at[pl.ds(0, R * HIST)], bufs[1], sem).wait()
        pltpu.make_async_copy(num_hbm.at[pl.ds(0, R * L)], bufs[2], sem).wait()
        for k in range(N_SPARSE):
            pltpu.make_async_copy(embs[k].at[spi_v.at[pl.ds(0, R)]],
                                  bufs[3].at[pl.ds(k * R, R)], sem).wait()

    bufsA = (b0A, b1A, numA, gA)
    bufsB = (b0B, b1B, numB, gB)
    stage(0, bufsA, semA)
    pltpu.sync_copy(t0_hbm, tbl0_v)
    pltpu.sync_copy(t1_hbm, tbl1_v)

    def pool(beh_v, gbase):
        tbl_v = tbl0_v if beh_v is b0A or beh_v is b0B else tbl1_v
        def body(h, accs):
            idx = plsc.load_gather(beh_v, [iotaH + (gbase + h)])
            out = []
            for w in range(NW):
                lo, hi = unpack(plsc.load_gather(tbl_v, [idx, cw[w]]))
                out.append(accs[2 * w] + lo)
                out.append(accs[2 * w + 1] + hi)
            return tuple(out)
        return plsc.parallel_loop(0, HIST, carry=(zero,) * EMB_DIM,
                                  unroll=4)(body)

    def compute(chunk, bufs):
        b0_v, b1_v, num_v, g_v = bufs
        def nbody(r, c):
            src = plsc.load_gather(num_v, [iota + r * L])
            plsc.store_scatter(out_v, [iota + r * 221], src, mask=m13)
            return c
        lax.fori_loop(0, R, nbody, 0)
        for g in range(R // L):
            dst_base = iota221 + g * L * 221
            for coff, beh_v in ((205, b0_v), (213, b1_v)):
                accs = pool(beh_v, g * L * HIST)
                for d in range(EMB_DIM):
                    plsc.store_scatter(out_v, [dst_base + (coff + d)], accs[d])
        for k in range(N_SPARSE):
            def sbody(j, c, k=k):
                src = plsc.load_gather(g_v, [lane_r + (k * R + j * 2), lane_d])
                dst = pat221 + (j * 442 + 13 + 8 * k)
                plsc.store_scatter(out_v, [dst], src)
                return c
            lax.fori_loop(0, R // 2, sbody, 0)
        row0 = base_row + chunk * R
        pltpu.sync_copy(out_v, out_hbm.at[pl.ds(row0 * N_COLS, R * N_COLS)])

    def pair_body(t, carry):
        c0 = 2 * t
        stage(c0 + 1, bufsB, semB)
        drain(bufsA, semA)
        compute(c0, bufsA)
        nxt = jnp.minimum(c0 + 2, N_CHUNKS - 1)
        stage(nxt, bufsA, semA)
        drain(bufsB, semB)
        compute(c0 + 1, bufsB)
        return carry

    lax.fori_loop(0, N_CHUNKS // 2, pair_body, 0)
    drain(bufsA, semA)


_sc_kernel = functools.partial(
    pl.kernel,
    out_type=jax.ShapeDtypeStruct((B * N_COLS,), jnp.float32),
    mesh=plsc.VectorSubcoreMesh(core_axis_name="c", subcore_axis_name="s"),
    compiler_params=pltpu.CompilerParams(needs_layout_passes=False,
                                         use_tc_tiling_on_sc=False),
    scratch_types=[
        pltpu.VMEM((NUM_BINS, NW), jnp.int32),
        pltpu.VMEM((NUM_BINS, NW), jnp.int32),
        pltpu.VMEM((R * L,), jnp.float32),
        pltpu.VMEM((R * L,), jnp.float32),
        pltpu.VMEM((R * HIST,), jnp.int32),
        pltpu.VMEM((R * HIST,), jnp.int32),
        pltpu.VMEM((R * HIST,), jnp.int32),
        pltpu.VMEM((R * HIST,), jnp.int32),
        pltpu.VMEM((N_SPARSE * ROWS_PER_WORKER,), jnp.int32),
        pltpu.VMEM((N_SPARSE * R, EMB_DIM), jnp.float32),
        pltpu.VMEM((N_SPARSE * R, EMB_DIM), jnp.float32),
        pltpu.VMEM((R * N_COLS,), jnp.float32),
        pltpu.SemaphoreType.DMA,
        pltpu.SemaphoreType.DMA,
    ],
)(_sc_body)


def kernel(log2_0, log2_1, log2_2, log2_3, log2_4, log2_5, log10_0, log10_1, log10_2, log10_3, log10_4, log10_5, log10_6, sparse_0, sparse_1, sparse_2, sparse_3, sparse_4, sparse_5, sparse_6, sparse_7, sparse_8, sparse_9, sparse_10, sparse_11, sparse_12, sparse_13, sparse_14, sparse_15, sparse_16, sparse_17, sparse_18, sparse_19, sparse_20, sparse_21, sparse_22, sparse_23, beh_0, beh_1, emb_sparse_0, emb_sparse_1, emb_sparse_2, emb_sparse_3, emb_sparse_4, emb_sparse_5, emb_sparse_6, emb_sparse_7, emb_sparse_8, emb_sparse_9, emb_sparse_10, emb_sparse_11, emb_sparse_12, emb_sparse_13, emb_sparse_14, emb_sparse_15, emb_sparse_16, emb_sparse_17, emb_sparse_18, emb_sparse_19, emb_sparse_20, emb_sparse_21, emb_sparse_22, emb_sparse_23, emb_beh_0, emb_beh_1):
    logs = [log2_0, log2_1, log2_2, log2_3, log2_4, log2_5,
            log10_0, log10_1, log10_2, log10_3, log10_4, log10_5, log10_6]
    sparse = [sparse_0, sparse_1, sparse_2, sparse_3, sparse_4, sparse_5,
              sparse_6, sparse_7, sparse_8, sparse_9, sparse_10, sparse_11,
              sparse_12, sparse_13, sparse_14, sparse_15, sparse_16, sparse_17,
              sparse_18, sparse_19, sparse_20, sparse_21, sparse_22, sparse_23]
    embs = [emb_sparse_0, emb_sparse_1, emb_sparse_2, emb_sparse_3,
            emb_sparse_4, emb_sparse_5, emb_sparse_6, emb_sparse_7,
            emb_sparse_8, emb_sparse_9, emb_sparse_10, emb_sparse_11,
            emb_sparse_12, emb_sparse_13, emb_sparse_14, emb_sparse_15,
            emb_sparse_16, emb_sparse_17, emb_sparse_18, emb_sparse_19,
            emb_sparse_20, emb_sparse_21, emb_sparse_22, emb_sparse_23]

    xpad = jnp.concatenate(logs + [jnp.zeros((B, 3), jnp.float32)], axis=1)
    scale16 = [1.0 / math.log(2.0)] * 6 + [1.0 / math.log(10.0)] * 7 + [0.0] * 3
    scale_row = jnp.tile(jnp.asarray(scale16, jnp.float32), 8)[None, :]
    num = _numeric_tc(xpad.reshape(B * L // 128, 128),
                      jnp.broadcast_to(scale_row, (B * L // 128, 128)))

    spi = jnp.stack([s.astype(jnp.int32).reshape(B) for s in sparse], axis=0)
    out_flat = _sc_kernel(
        num.reshape(-1),
        spi.reshape(-1),
        beh_0.astype(jnp.int32).reshape(-1),
        beh_1.astype(jnp.int32).reshape(-1),
        _pack_bf16(emb_beh_0), _pack_bf16(emb_beh_1),
        *embs,
    )
    return out_flat.reshape(B, N_COLS)
